# baseline (speedup 1.0000x reference)
import functools

import jax
import jax.numpy as jnp
from jax import lax
from jax.experimental import pallas as pl
from jax.experimental.pallas import tpu as pltpu
from jax.experimental.pallas import tpu_sc as plsc

N_USERS = 100000
N_ITEMS = 100000
K = 128
B = 4096
U_MEAN = 3.5

NC = 2
NS = 16
L = 16
NW = NC * NS
BPW = B // NW

_mesh = plsc.VectorSubcoreMesh(core_axis_name="c", subcore_axis_name="s")


@functools.partial(
    pl.kernel,
    mesh=_mesh,
    out_type=jax.ShapeDtypeStruct((B,), jnp.float32),
    scratch_types=[
        pltpu.VMEM((BPW,), jnp.int32),
        pltpu.VMEM((BPW,), jnp.int32),
        pltpu.VMEM((BPW,), jnp.int32),
        pltpu.VMEM((BPW,), jnp.float32),
        pltpu.VMEM((BPW,), jnp.float32),
        pltpu.VMEM((BPW,), jnp.float32),
        pltpu.VMEM((K * BPW,), jnp.int32),
        pltpu.VMEM((K * BPW,), jnp.int32),
        pltpu.VMEM((K * BPW,), jnp.float32),
        pltpu.VMEM((K * BPW,), jnp.float32),
        pltpu.VMEM((BPW,), jnp.float32),
        pltpu.SemaphoreType.DMA,
        pltpu.SemaphoreType.DMA,
    ],
)
def _svdpp(u_h, i_h, bu_h, bi_h, puf_h, qif_h, ru_h, out_h,
           u_v, i_v, uk_v, bu_v, bi_v, ru_v, pidx_v, qidx_v, pt_v, qt_v,
           res_v, sem_a, sem_b):
    wid = lax.axis_index("s") * NC + lax.axis_index("c")
    base = wid * BPW

    pltpu.sync_copy(u_h.at[pl.ds(base, BPW)], u_v)
    pltpu.sync_copy(i_h.at[pl.ds(base, BPW)], i_v)

    cp_bu = pltpu.async_copy(bu_h.at[u_v], bu_v, sem_a)
    cp_bi = pltpu.async_copy(bi_h.at[i_v], bi_v, sem_a)
    cp_ru = pltpu.async_copy(ru_h.at[u_v], ru_v, sem_a)

    NCH = BPW // L
    for c in range(NCH):
        sl = pl.ds(c * L, L)
        uk_v[sl] = u_v[sl] * K

    def gen(k, carry):
        koff = k * N_ITEMS
        row = k * BPW
        for c in range(NCH):
            sl = pl.ds(c * L, L)
            fsl = pl.ds(row + c * L, L)
            pidx_v[fsl] = uk_v[sl] + k
            qidx_v[fsl] = i_v[sl] + koff
        return carry

    lax.fori_loop(0, K, gen, 0)

    cp_pt = pltpu.async_copy(puf_h.at[pidx_v], pt_v, sem_b)
    cp_qt = pltpu.async_copy(qif_h.at[qidx_v], qt_v, sem_b)

    cp_bu.wait()
    cp_bi.wait()
    cp_ru.wait()
    cp_pt.wait()
    cp_qt.wait()

    def group_body(g, carry):
        sl = pl.ds(g * L, L)
        ruv = ru_v[sl]

        def dot_body(k, acc):
            fsl = pl.ds(k * BPW + g * L, L)
            return acc + (pt_v[fsl] + ruv) * qt_v[fsl]

        acc = lax.fori_loop(0, K, dot_body, jnp.zeros((L,), jnp.float32))
        res_v[sl] = bu_v[sl] + bi_v[sl] + U_MEAN + acc
        return carry

    lax.fori_loop(0, NCH, group_body, 0)

    pltpu.sync_copy(res_v, out_h.at[pl.ds(base, BPW)])


def kernel(u, i, bu, bi, pu, qi, Ru):
    return _svdpp(
        u.astype(jnp.int32),
        i.astype(jnp.int32),
        bu,
        bi,
        pu.reshape(-1),
        qi.reshape(-1),
        Ru.reshape(-1),
    )


# speedup vs baseline: 1.3248x; 1.3248x over previous
import functools

import jax
import jax.numpy as jnp
from jax import lax
from jax.experimental import pallas as pl
from jax.experimental.pallas import tpu as pltpu
from jax.experimental.pallas import tpu_sc as plsc

N_USERS = 100000
N_ITEMS = 100000
K = 128
B = 4096
U_MEAN = 3.5

NC = 2
NS = 16
L = 16
NW = NC * NS
BPW = B // NW
NCH = BPW // L

_mesh = plsc.VectorSubcoreMesh(core_axis_name="c", subcore_axis_name="s")


@functools.partial(
    pl.kernel,
    mesh=_mesh,
    out_type=jax.ShapeDtypeStruct((B,), jnp.float32),
    scratch_types=[
        pltpu.VMEM((BPW,), jnp.int32),
        pltpu.VMEM((BPW,), jnp.int32),
        pltpu.VMEM((BPW,), jnp.float32),
        pltpu.VMEM((BPW,), jnp.float32),
        pltpu.VMEM((BPW,), jnp.float32),
        pltpu.VMEM((BPW, K), jnp.float32),
        pltpu.VMEM((K * BPW,), jnp.int32),
        pltpu.VMEM((K * BPW,), jnp.float32),
        pltpu.VMEM((BPW,), jnp.float32),
        pltpu.SemaphoreType.DMA,
        pltpu.SemaphoreType.DMA,
    ],
)
def _svdpp(u_h, i_h, bu_h, bi_h, pu_h, qif_h, ru_h, out_h,
           u_v, i_v, bu_v, bi_v, ru_v, pu_v, qidx_v, qv_v, res_v,
           sem_a, sem_b):
    wid = lax.axis_index("s") * NC + lax.axis_index("c")
    base = wid * BPW

    pltpu.sync_copy(u_h.at[pl.ds(base, BPW)], u_v)
    pltpu.sync_copy(i_h.at[pl.ds(base, BPW)], i_v)

    cp_pu = pltpu.async_copy(pu_h.at[u_v], pu_v, sem_a)
    cp_bu = pltpu.async_copy(bu_h.at[u_v], bu_v, sem_a)
    cp_bi = pltpu.async_copy(bi_h.at[i_v], bi_v, sem_a)
    cp_ru = pltpu.async_copy(ru_h.at[u_v], ru_v, sem_a)

    kstep = lax.iota(jnp.int32, L) * N_ITEMS

    def gen(cc, carry):
        iv = i_v[pl.ds(cc * L, L)]
        for jj in range(L):
            j = cc * L + jj
            ibc = lax.broadcast(iv[jj], (L,)) + kstep
            for c in range(K // L):
                qidx_v[pl.ds(j * K + c * L, L)] = ibc + (c * L * N_ITEMS)
        return carry

    lax.fori_loop(0, NCH, gen, 0)

    cp_qv = pltpu.async_copy(qif_h.at[qidx_v], qv_v, sem_b)

    cp_pu.wait()
    cp_bu.wait()
    cp_bi.wait()
    cp_ru.wait()
    cp_qv.wait()

    lane = lax.iota(jnp.int32, L)
    zero = jnp.zeros((L,), jnp.float32)

    def group_body(g, carry):
        sl = pl.ds(g * L, L)
        ruv = ru_v[sl]
        acc = zero
        for jj in range(L):
            j = g * L + jj
            rbc = lax.broadcast(ruv[jj], (L,))
            pa = zero
            for c in range(K // L):
                csl = pl.ds(c * L, L)
                pa = pa + (pu_v[j, csl] + rbc) * qv_v[pl.ds(j * K + c * L, L)]
            s01 = pa[0] + pa[1]
            s23 = pa[2] + pa[3]
            s45 = pa[4] + pa[5]
            s67 = pa[6] + pa[7]
            s89 = pa[8] + pa[9]
            sab = pa[10] + pa[11]
            scd = pa[12] + pa[13]
            sef = pa[14] + pa[15]
            s = ((s01 + s23) + (s45 + s67)) + ((s89 + sab) + (scd + sef))
            acc = jnp.where(lane == jj, lax.broadcast(s, (L,)), acc)
        res_v[sl] = bu_v[sl] + bi_v[sl] + U_MEAN + acc
        return carry

    lax.fori_loop(0, NCH, group_body, 0)

    pltpu.sync_copy(res_v, out_h.at[pl.ds(base, BPW)])


def kernel(u, i, bu, bi, pu, qi, Ru):
    return _svdpp(
        u.astype(jnp.int32),
        i.astype(jnp.int32),
        bu,
        bi,
        pu,
        qi.reshape(-1),
        Ru.reshape(-1),
    )


# speedup vs baseline: 7.1478x; 5.3954x over previous
import functools

import jax
import jax.numpy as jnp
from jax import lax
from jax.experimental import pallas as pl
from jax.experimental.pallas import tpu as pltpu
from jax.experimental.pallas import tpu_sc as plsc

N_USERS = 100000
N_ITEMS = 100000
K = 128
B = 4096
U_MEAN = 3.5

NC = 2
NS = 16
L = 16
NW = NC * NS
BPW = B // NW
NCH = BPW // L
HALF = BPW // 2

_mesh = plsc.VectorSubcoreMesh(core_axis_name="c", subcore_axis_name="s")


@functools.partial(
    pl.kernel,
    mesh=_mesh,
    out_type=jax.ShapeDtypeStruct((B,), jnp.float32),
    scratch_types=[
        pltpu.VMEM((BPW,), jnp.int32),
        pltpu.VMEM((BPW,), jnp.int32),
        pltpu.VMEM((BPW,), jnp.float32),
        pltpu.VMEM((BPW,), jnp.float32),
        pltpu.VMEM((BPW,), jnp.float32),
        pltpu.VMEM((BPW, K), jnp.float32),
        pltpu.VMEM((BPW, K), jnp.float32),
        pltpu.VMEM((BPW,), jnp.float32),
        pltpu.SemaphoreType.DMA,
        pltpu.SemaphoreType.DMA,
        pltpu.SemaphoreType.DMA,
    ],
)
def _svdpp(u_h, i_h, bu_h, bi_h, pu_h, qit_h, ru_h, out_h,
           u_v, i_v, bu_v, bi_v, ru_v, pu_v, qt_v, res_v,
           sem_a, sem_b, sem_m):
    wid = lax.axis_index("s") * NC + lax.axis_index("c")
    base = wid * BPW

    pltpu.sync_copy(u_h.at[pl.ds(base, BPW)], u_v)
    pltpu.sync_copy(i_h.at[pl.ds(base, BPW)], i_v)

    lo = pl.ds(0, HALF)
    hi = pl.ds(HALF, HALF)
    cp_pu_a = pltpu.async_copy(pu_h.at[u_v.at[lo]], pu_v.at[lo], sem_a)
    cp_qt_a = pltpu.async_copy(qit_h.at[i_v.at[lo]], qt_v.at[lo], sem_a)
    cp_pu_b = pltpu.async_copy(pu_h.at[u_v.at[hi]], pu_v.at[hi], sem_b)
    cp_qt_b = pltpu.async_copy(qit_h.at[i_v.at[hi]], qt_v.at[hi], sem_b)
    cp_bu = pltpu.async_copy(bu_h.at[u_v], bu_v, sem_m)
    cp_bi = pltpu.async_copy(bi_h.at[i_v], bi_v, sem_m)
    cp_ru = pltpu.async_copy(ru_h.at[u_v], ru_v, sem_m)

    lane = lax.iota(jnp.int32, L)
    zero = jnp.zeros((L,), jnp.float32)

    def group_body(g, carry):
        sl = pl.ds(g * L, L)
        ruv = ru_v[sl]
        acc = zero
        for jj in range(L):
            j = g * L + jj
            rbc = lax.broadcast(ruv[jj], (L,))
            pa = zero
            for c in range(K // L):
                csl = pl.ds(c * L, L)
                pa = pa + (pu_v[j, csl] + rbc) * qt_v[j, csl]
            s01 = pa[0] + pa[1]
            s23 = pa[2] + pa[3]
            s45 = pa[4] + pa[5]
            s67 = pa[6] + pa[7]
            s89 = pa[8] + pa[9]
            sab = pa[10] + pa[11]
            scd = pa[12] + pa[13]
            sef = pa[14] + pa[15]
            s = ((s01 + s23) + (s45 + s67)) + ((s89 + sab) + (scd + sef))
            acc = jnp.where(lane == jj, lax.broadcast(s, (L,)), acc)
        res_v[sl] = bu_v[sl] + bi_v[sl] + U_MEAN + acc
        return carry

    cp_ru.wait()
    cp_bu.wait()
    cp_bi.wait()
    cp_pu_a.wait()
    cp_qt_a.wait()
    lax.fori_loop(0, NCH // 2, group_body, 0)
    cp_pu_b.wait()
    cp_qt_b.wait()
    lax.fori_loop(NCH // 2, NCH, group_body, 0)

    pltpu.sync_copy(res_v, out_h.at[pl.ds(base, BPW)])


def kernel(u, i, bu, bi, pu, qi, Ru):
    return _svdpp(
        u.astype(jnp.int32),
        i.astype(jnp.int32),
        bu,
        bi,
        pu,
        qi.T,
        Ru.reshape(-1),
    )


# speedup vs baseline: 7.5415x; 1.0551x over previous
import functools

import jax
import jax.numpy as jnp
from jax import lax
from jax.experimental import pallas as pl
from jax.experimental.pallas import tpu as pltpu
from jax.experimental.pallas import tpu_sc as plsc

N_USERS = 100000
N_ITEMS = 100000
K = 128
B = 4096
U_MEAN = 3.5

NC = 2
NS = 16
L = 16
NW = NC * NS
BPW = B // NW
NCH = BPW // L
HALF = BPW // 2

_mesh = plsc.VectorSubcoreMesh(core_axis_name="c", subcore_axis_name="s")


@functools.partial(
    pl.kernel,
    mesh=_mesh,
    out_type=jax.ShapeDtypeStruct((B,), jnp.float32),
    scratch_types=[
        pltpu.VMEM((BPW,), jnp.int32),
        pltpu.VMEM((BPW,), jnp.int32),
        pltpu.VMEM((BPW,), jnp.float32),
        pltpu.VMEM((BPW,), jnp.float32),
        pltpu.VMEM((BPW,), jnp.float32),
        pltpu.VMEM((BPW, K), jnp.float32),
        pltpu.VMEM((BPW, K), jnp.float32),
        pltpu.VMEM((BPW,), jnp.float32),
        pltpu.SemaphoreType.DMA,
    ],
)
def _svdpp(u_h, i_h, bu_h, bi_h, pu_h, qit_h, ru_h, out_h,
           u_v, i_v, bu_v, bi_v, ru_v, pu_v, qt_v, res_v, sem):
    wid = lax.axis_index("s") * NC + lax.axis_index("c")
    base = wid * BPW

    cp_u = pltpu.async_copy(u_h.at[pl.ds(base, BPW)], u_v, sem)
    cp_i = pltpu.async_copy(i_h.at[pl.ds(base, BPW)], i_v, sem)
    cp_u.wait()
    cp_i.wait()

    cp_pu = pltpu.async_copy(pu_h.at[u_v], pu_v, sem)
    cp_qt = pltpu.async_copy(qit_h.at[i_v], qt_v, sem)
    cp_bu = pltpu.async_copy(bu_h.at[u_v], bu_v, sem)
    cp_bi = pltpu.async_copy(bi_h.at[i_v], bi_v, sem)
    cp_ru = pltpu.async_copy(ru_h.at[u_v], ru_v, sem)

    lane = lax.iota(jnp.int32, L)
    zero = jnp.zeros((L,), jnp.float32)

    def group_body(g, carry):
        sl = pl.ds(g * L, L)
        ruv = ru_v[sl]
        acc = zero
        for jj in range(L):
            j = g * L + jj
            rbc = lax.broadcast(ruv[jj], (L,))
            pa = zero
            for c in range(K // L):
                csl = pl.ds(c * L, L)
                pa = pa + (pu_v[j, csl] + rbc) * qt_v[j, csl]
            s01 = pa[0] + pa[1]
            s23 = pa[2] + pa[3]
            s45 = pa[4] + pa[5]
            s67 = pa[6] + pa[7]
            s89 = pa[8] + pa[9]
            sab = pa[10] + pa[11]
            scd = pa[12] + pa[13]
            sef = pa[14] + pa[15]
            s = ((s01 + s23) + (s45 + s67)) + ((s89 + sab) + (scd + sef))
            acc = jnp.where(lane == jj, lax.broadcast(s, (L,)), acc)
        res_v[sl] = bu_v[sl] + bi_v[sl] + U_MEAN + acc
        return carry

    cp_pu.wait()
    cp_qt.wait()
    cp_bu.wait()
    cp_bi.wait()
    cp_ru.wait()
    lax.fori_loop(0, NCH, group_body, 0)

    pltpu.sync_copy(res_v, out_h.at[pl.ds(base, BPW)])


def kernel(u, i, bu, bi, pu, qi, Ru):
    return _svdpp(
        u.astype(jnp.int32),
        i.astype(jnp.int32),
        bu,
        bi,
        pu,
        qi.T,
        Ru.reshape(-1),
    )


# speedup vs baseline: 7.5604x; 1.0025x over previous
import functools

import jax
import jax.numpy as jnp
from jax import lax
from jax.experimental import pallas as pl
from jax.experimental.pallas import tpu as pltpu
from jax.experimental.pallas import tpu_sc as plsc

N_USERS = 100000
N_ITEMS = 100000
K = 128
B = 4096
U_MEAN = 3.5

NC = 2
NS = 16
L = 16
NW = NC * NS
BPW = B // NW
NCH = BPW // L
HALF = BPW // 2

_mesh = plsc.VectorSubcoreMesh(core_axis_name="c", subcore_axis_name="s")


@functools.partial(
    pl.kernel,
    mesh=_mesh,
    out_type=jax.ShapeDtypeStruct((B,), jnp.float32),
    scratch_types=[
        pltpu.VMEM((BPW,), jnp.int32),
        pltpu.VMEM((BPW,), jnp.int32),
        pltpu.VMEM((BPW,), jnp.float32),
        pltpu.VMEM((BPW,), jnp.float32),
        pltpu.VMEM((BPW,), jnp.float32),
        pltpu.VMEM((BPW, K), jnp.float32),
        pltpu.VMEM((BPW, K), jnp.float32),
        pltpu.VMEM((BPW,), jnp.float32),
        pltpu.SemaphoreType.DMA,
    ],
)
def _svdpp(u_h, i_h, bu_h, bi_h, pu_h, qit_h, ru_h, out_h,
           u_v, i_v, bu_v, bi_v, ru_v, pu_v, qt_v, res_v, sem):
    wid = lax.axis_index("s") * NC + lax.axis_index("c")
    base = wid * BPW

    cp_u = pltpu.async_copy(u_h.at[pl.ds(base, BPW)], u_v, sem)
    cp_i = pltpu.async_copy(i_h.at[pl.ds(base, BPW)], i_v, sem)
    cp_u.wait()
    cp_i.wait()

    cp_pu = pltpu.async_copy(pu_h.at[u_v], pu_v, sem)
    cp_qt = pltpu.async_copy(qit_h.at[i_v], qt_v, sem)
    cp_bu = pltpu.async_copy(bu_h.at[u_v], bu_v, sem)
    cp_bi = pltpu.async_copy(bi_h.at[i_v], bi_v, sem)
    cp_ru = pltpu.async_copy(ru_h.at[u_v], ru_v, sem)

    lane = lax.iota(jnp.int32, L)
    zero = jnp.zeros((L,), jnp.float32)

    cp_pu.wait()
    cp_qt.wait()
    cp_bu.wait()
    cp_bi.wait()
    cp_ru.wait()

    @plsc.parallel_loop(0, NCH)
    def group_body(g):
        sl = pl.ds(g * L, L)
        ruv = ru_v[sl]
        acc = zero
        for jj in range(L):
            j = g * L + jj
            rbc = lax.broadcast(ruv[jj], (L,))
            pa = zero
            for c in range(K // L):
                csl = pl.ds(c * L, L)
                pa = pa + (pu_v[j, csl] + rbc) * qt_v[j, csl]
            s01 = pa[0] + pa[1]
            s23 = pa[2] + pa[3]
            s45 = pa[4] + pa[5]
            s67 = pa[6] + pa[7]
            s89 = pa[8] + pa[9]
            sab = pa[10] + pa[11]
            scd = pa[12] + pa[13]
            sef = pa[14] + pa[15]
            s = ((s01 + s23) + (s45 + s67)) + ((s89 + sab) + (scd + sef))
            acc = jnp.where(lane == jj, lax.broadcast(s, (L,)), acc)
        res_v[sl] = bu_v[sl] + bi_v[sl] + U_MEAN + acc


    pltpu.sync_copy(res_v, out_h.at[pl.ds(base, BPW)])


def kernel(u, i, bu, bi, pu, qi, Ru):
    return _svdpp(
        u.astype(jnp.int32),
        i.astype(jnp.int32),
        bu,
        bi,
        pu,
        qi.T,
        Ru.reshape(-1),
    )


# speedup vs baseline: 8.1288x; 1.0752x over previous
import functools

import jax
import jax.numpy as jnp
from jax import lax
from jax.experimental import pallas as pl
from jax.experimental.pallas import tpu as pltpu
from jax.experimental.pallas import tpu_sc as plsc

N_USERS = 100000
N_ITEMS = 100000
K = 128
B = 4096
U_MEAN = 3.5

NC = 2
NS = 16
L = 16
NW = NC * NS
BPW = B // NW
NCH = BPW // L
HALF = BPW // 2

_mesh = plsc.VectorSubcoreMesh(core_axis_name="c", subcore_axis_name="s")


@functools.partial(
    pl.kernel,
    mesh=_mesh,
    out_type=jax.ShapeDtypeStruct((B,), jnp.float32),
    scratch_types=[
        pltpu.VMEM((BPW,), jnp.int32),
        pltpu.VMEM((BPW,), jnp.int32),
        pltpu.VMEM((BPW,), jnp.float32),
        pltpu.VMEM((BPW,), jnp.float32),
        pltpu.VMEM((BPW,), jnp.float32),
        pltpu.VMEM((BPW, K), jnp.float32),
        pltpu.VMEM((BPW, K), jnp.float32),
        pltpu.VMEM((BPW,), jnp.float32),
        pltpu.SemaphoreType.DMA,
    ],
)
def _svdpp(u_h, i_h, bu_h, bi_h, pu_h, qit_h, ru_h, out_h,
           u_v, i_v, bu_v, bi_v, ru_v, pu_v, qt_v, res_v, sem):
    wid = lax.axis_index("s") * NC + lax.axis_index("c")
    base = wid * BPW

    cp_u = pltpu.async_copy(u_h.at[pl.ds(base, BPW)], u_v, sem)
    cp_i = pltpu.async_copy(i_h.at[pl.ds(base, BPW)], i_v, sem)
    cp_u.wait()
    cp_i.wait()

    cp_pu = pltpu.async_copy(pu_h.at[u_v], pu_v, sem)
    cp_qt = pltpu.async_copy(qit_h.at[i_v], qt_v, sem)
    cp_bu = pltpu.async_copy(bu_h.at[u_v], bu_v, sem)
    cp_bi = pltpu.async_copy(bi_h.at[i_v], bi_v, sem)
    cp_ru = pltpu.async_copy(ru_h.at[u_v], ru_v, sem)

    lane = lax.iota(jnp.int32, L)
    zero = jnp.zeros((L,), jnp.float32)
    perm8 = (lane + 8) & (L - 1)

    cp_pu.wait()
    cp_qt.wait()
    cp_bu.wait()
    cp_bi.wait()
    cp_ru.wait()

    @plsc.parallel_loop(0, NCH)
    def group_body(g):
        sl = pl.ds(g * L, L)
        ruv = ru_v[sl]
        acc = zero
        for jj in range(L):
            j = g * L + jj
            rbc = lax.broadcast(ruv[jj], (L,))
            pa = zero
            for c in range(K // L):
                csl = pl.ds(c * L, L)
                pa = pa + (pu_v[j, csl] + rbc) * qt_v[j, csl]
            pa = pa + pa.at[perm8].get(mode="promise_in_bounds")
            s01 = pa[0] + pa[1]
            s23 = pa[2] + pa[3]
            s45 = pa[4] + pa[5]
            s67 = pa[6] + pa[7]
            s = (s01 + s23) + (s45 + s67)
            acc = jnp.where(lane == jj, lax.broadcast(s, (L,)), acc)
        res_v[sl] = bu_v[sl] + bi_v[sl] + U_MEAN + acc


    pltpu.sync_copy(res_v, out_h.at[pl.ds(base, BPW)])


def kernel(u, i, bu, bi, pu, qi, Ru):
    return _svdpp(
        u.astype(jnp.int32),
        i.astype(jnp.int32),
        bu,
        bi,
        pu,
        qi.T,
        Ru.reshape(-1),
    )
